# baseline (speedup 1.0000x reference)
import jax
import jax.numpy as jnp
from jax import lax
from jax.experimental import pallas as pl
from jax.experimental.pallas import tpu as pltpu
from jax.experimental.pallas import tpu_sc as plsc

_B, _T, _U, _V = 4, 512, 50, 256
_UH = _U // 2
_TQ = _T // 4
_LANES = 16
_NVC = _V // _LANES


def _logsoftmax_body(tn_ref, pn_ref, tn_ls_ref, pn_ls_ref):
    tn = tn_ref[0]
    tn_max = jnp.max(tn, axis=-1, keepdims=True)
    tn_ls_ref[0] = tn - tn_max - jnp.log(
        jnp.sum(jnp.exp(tn - tn_max), axis=-1, keepdims=True))
    pn = pn_ref[0]
    pn_max = jnp.max(pn, axis=-1, keepdims=True)
    pn_ls = pn - pn_max - jnp.log(
        jnp.sum(jnp.exp(pn - pn_max), axis=-1, keepdims=True))
    col = jax.lax.broadcasted_iota(jnp.int32, pn_ls.shape, 1)
    pn_ls_ref[0] = jnp.where(col == 0, 0.0, pn_ls)


def _log_softmaxes(tn_out, pn_out):
    return pl.pallas_call(
        _logsoftmax_body,
        grid=(_B,),
        in_specs=[
            pl.BlockSpec((1, _T, _V), lambda b: (b, 0, 0)),
            pl.BlockSpec((1, _U, _V), lambda b: (b, 0, 0)),
        ],
        out_specs=[
            pl.BlockSpec((1, _T, _V), lambda b: (b, 0, 0)),
            pl.BlockSpec((1, _U, _V), lambda b: (b, 0, 0)),
        ],
        out_shape=[
            jax.ShapeDtypeStruct((_B, _T, _V), tn_out.dtype),
            jax.ShapeDtypeStruct((_B, _U, _V), pn_out.dtype),
        ],
    )(tn_out, pn_out)


def _expand_compute(ob, pn_v, tn_v, u):
    pnregs = tuple(pn_v[u, pl.ds(vc * _LANES, _LANES)] for vc in range(_NVC))

    def tbody(t, regs):
        for vc in range(_NVC):
            sl = pl.ds(vc * _LANES, _LANES)
            ob[t, sl] = tn_v[t, sl] + regs[vc]
        return regs

    lax.fori_loop(0, _TQ, tbody, pnregs)


def _expand_body(tn_ls, pn_ls, out, pn_v, tn_v, ob0, ob1, sem0, sem1):
    wid = lax.axis_index("s") * 2 + lax.axis_index("c")
    b = wid // 8
    r = wid % 8
    u0 = (r // 4) * _UH
    t0 = (r % 4) * _TQ

    pltpu.sync_copy(pn_ls.at[b], pn_v)
    pltpu.sync_copy(tn_ls.at[b, pl.ds(t0, _TQ)], tn_v)

    def upair(j, carry):
        u_a = 2 * j
        u_b = 2 * j + 1

        @pl.when(j > 0)
        def _():
            pltpu.make_async_copy(
                ob0, out.at[b, u0 + u_a - 2, pl.ds(t0, _TQ)], sem0).wait()

        @pl.when(j == 0)
        def _():
            _expand_compute(ob0, pn_v, tn_v, u0 + u_a)
        pltpu.make_async_copy(
            ob0, out.at[b, u0 + u_a, pl.ds(t0, _TQ)], sem0).start()

        @pl.when(j > 0)
        def _():
            pltpu.make_async_copy(
                ob1, out.at[b, u0 + u_b - 2, pl.ds(t0, _TQ)], sem1).wait()

        @pl.when(j == 0)
        def _():
            _expand_compute(ob1, pn_v, tn_v, u0 + u_b)
        pltpu.make_async_copy(
            ob1, out.at[b, u0 + u_b, pl.ds(t0, _TQ)], sem1).start()
        return carry

    lax.fori_loop(0, _UH // 2, upair, 0)

    u_t = _UH - 1
    pltpu.make_async_copy(
        ob0, out.at[b, u0 + u_t - 2, pl.ds(t0, _TQ)], sem0).wait()
    pltpu.make_async_copy(
        ob0, out.at[b, u0 + u_t, pl.ds(t0, _TQ)], sem0).start()
    pltpu.make_async_copy(
        ob0, out.at[b, u0 + u_t, pl.ds(t0, _TQ)], sem0).wait()
    pltpu.make_async_copy(
        ob1, out.at[b, u0 + u_t - 1, pl.ds(t0, _TQ)], sem1).wait()


def _expand(tn_ls, pn_ls):
    mesh = plsc.VectorSubcoreMesh(core_axis_name="c", subcore_axis_name="s")
    run = pl.kernel(
        _expand_body,
        out_type=jax.ShapeDtypeStruct((_B, _U, _T, _V), tn_ls.dtype),
        mesh=mesh,
        scratch_types=[
            pltpu.VMEM((_U, _V), jnp.float32),
            pltpu.VMEM((_TQ, _V), jnp.float32),
            pltpu.VMEM((_TQ, _V), jnp.float32),
            pltpu.VMEM((_TQ, _V), jnp.float32),
            pltpu.SemaphoreType.DMA,
            pltpu.SemaphoreType.DMA,
        ],
    )
    return run(tn_ls, pn_ls)


def kernel(tn_out, pn_out):
    tn_ls, pn_ls = _log_softmaxes(tn_out, pn_out)
    out_butv = _expand(tn_ls, pn_ls)
    return jnp.swapaxes(out_butv, 1, 2)


# speedup vs baseline: 1.0021x; 1.0021x over previous
import jax
import jax.numpy as jnp
from jax import lax
from jax.experimental import pallas as pl
from jax.experimental.pallas import tpu as pltpu
from jax.experimental.pallas import tpu_sc as plsc

_B, _T, _U, _V = 4, 512, 50, 256
_UH = _U // 2
_TQ = _T // 4
_LANES = 16
_NVC = _V // _LANES


def _logsoftmax_body(tn_ref, pn_ref, tn_ls_ref, pn_ls_ref):
    tn = tn_ref[0]
    tn_max = jnp.max(tn, axis=-1, keepdims=True)
    tn_ls_ref[0] = tn - tn_max - jnp.log(
        jnp.sum(jnp.exp(tn - tn_max), axis=-1, keepdims=True))
    pn = pn_ref[0]
    pn_max = jnp.max(pn, axis=-1, keepdims=True)
    pn_ls = pn - pn_max - jnp.log(
        jnp.sum(jnp.exp(pn - pn_max), axis=-1, keepdims=True))
    col = jax.lax.broadcasted_iota(jnp.int32, pn_ls.shape, 1)
    pn_ls_ref[0] = jnp.where(col == 0, 0.0, pn_ls)


def _log_softmaxes(tn_out, pn_out):
    return pl.pallas_call(
        _logsoftmax_body,
        grid=(_B,),
        in_specs=[
            pl.BlockSpec((1, _T, _V), lambda b: (b, 0, 0)),
            pl.BlockSpec((1, _U, _V), lambda b: (b, 0, 0)),
        ],
        out_specs=[
            pl.BlockSpec((1, _T, _V), lambda b: (b, 0, 0)),
            pl.BlockSpec((1, _U, _V), lambda b: (b, 0, 0)),
        ],
        out_shape=[
            jax.ShapeDtypeStruct((_B, _T, _V), tn_out.dtype),
            jax.ShapeDtypeStruct((_B, _U, _V), pn_out.dtype),
        ],
    )(tn_out, pn_out)


def _expand_compute(ob, pn_v, tn_v, u):
    pnregs = tuple(pn_v[u, pl.ds(vc * _LANES, _LANES)] for vc in range(_NVC))

    def tbody(t, regs):
        for vc in range(_NVC):
            sl = pl.ds(vc * _LANES, _LANES)
            ob[t, sl] = tn_v[t, sl] + regs[vc]
        return regs

    lax.fori_loop(0, _TQ, tbody, pnregs)


def _expand_body(tn_ls, pn_ls, out, pn_v, tn_v, ob0, ob1, sem0, sem1):
    wid = lax.axis_index("s") * 2 + lax.axis_index("c")
    b = wid // 8
    r = wid % 8
    u0 = (r // 4) * _UH
    t0 = (r % 4) * _TQ

    pltpu.sync_copy(pn_ls.at[b], pn_v)
    pltpu.sync_copy(tn_ls.at[b, pl.ds(t0, _TQ)], tn_v)

    def upair(j, carry):
        u_a = 2 * j
        u_b = 2 * j + 1

        th = _TQ // 2

        @pl.when(j > 0)
        def _():
            pltpu.make_async_copy(
                ob0, out.at[b, u0 + u_a - 2, pl.ds(t0, _TQ)], sem0).wait()

        @pl.when(j == 0)
        def _():
            _expand_compute(ob0, pn_v, tn_v, u0 + u_a)
        pltpu.make_async_copy(
            ob0.at[pl.ds(0, th)],
            out.at[b, u0 + u_a, pl.ds(t0, th)], sem0).start()
        pltpu.make_async_copy(
            ob0.at[pl.ds(th, th)],
            out.at[b, u0 + u_a, pl.ds(t0 + th, th)], sem0).start()

        @pl.when(j > 0)
        def _():
            pltpu.make_async_copy(
                ob1, out.at[b, u0 + u_b - 2, pl.ds(t0, _TQ)], sem1).wait()

        @pl.when(j == 0)
        def _():
            _expand_compute(ob1, pn_v, tn_v, u0 + u_b)
        pltpu.make_async_copy(
            ob1.at[pl.ds(0, th)],
            out.at[b, u0 + u_b, pl.ds(t0, th)], sem1).start()
        pltpu.make_async_copy(
            ob1.at[pl.ds(th, th)],
            out.at[b, u0 + u_b, pl.ds(t0 + th, th)], sem1).start()
        return carry

    lax.fori_loop(0, _UH // 2, upair, 0)

    u_t = _UH - 1
    pltpu.make_async_copy(
        ob0, out.at[b, u0 + u_t - 2, pl.ds(t0, _TQ)], sem0).wait()
    pltpu.make_async_copy(
        ob0, out.at[b, u0 + u_t, pl.ds(t0, _TQ)], sem0).start()
    pltpu.make_async_copy(
        ob0, out.at[b, u0 + u_t, pl.ds(t0, _TQ)], sem0).wait()
    pltpu.make_async_copy(
        ob1, out.at[b, u0 + u_t - 1, pl.ds(t0, _TQ)], sem1).wait()


def _expand(tn_ls, pn_ls):
    mesh = plsc.VectorSubcoreMesh(core_axis_name="c", subcore_axis_name="s")
    run = pl.kernel(
        _expand_body,
        out_type=jax.ShapeDtypeStruct((_B, _U, _T, _V), tn_ls.dtype),
        mesh=mesh,
        scratch_types=[
            pltpu.VMEM((_U, _V), jnp.float32),
            pltpu.VMEM((_TQ, _V), jnp.float32),
            pltpu.VMEM((_TQ, _V), jnp.float32),
            pltpu.VMEM((_TQ, _V), jnp.float32),
            pltpu.SemaphoreType.DMA,
            pltpu.SemaphoreType.DMA,
        ],
    )
    return run(tn_ls, pn_ls)


def kernel(tn_out, pn_out):
    tn_ls, pn_ls = _log_softmaxes(tn_out, pn_out)
    out_butv = _expand(tn_ls, pn_ls)
    return jnp.swapaxes(out_butv, 1, 2)


# speedup vs baseline: 1.0070x; 1.0049x over previous
import jax
import jax.numpy as jnp
from jax import lax
from jax.experimental import pallas as pl
from jax.experimental.pallas import tpu as pltpu
from jax.experimental.pallas import tpu_sc as plsc

_B, _T, _U, _V = 4, 512, 50, 256
_UH = _U // 2
_TQ = _T // 4
_LANES = 16
_NVC = _V // _LANES


def _logsoftmax_body(tn_ref, pn_ref, tn_ls_ref, pn_ls_ref):
    tn = tn_ref[0]
    tn_max = jnp.max(tn, axis=-1, keepdims=True)
    tn_ls_ref[0] = tn - tn_max - jnp.log(
        jnp.sum(jnp.exp(tn - tn_max), axis=-1, keepdims=True))
    pn = pn_ref[0]
    pn_max = jnp.max(pn, axis=-1, keepdims=True)
    pn_ls = pn - pn_max - jnp.log(
        jnp.sum(jnp.exp(pn - pn_max), axis=-1, keepdims=True))
    col = jax.lax.broadcasted_iota(jnp.int32, pn_ls.shape, 1)
    pn_ls_ref[0] = jnp.where(col == 0, 0.0, pn_ls)


def _log_softmaxes(tn_out, pn_out):
    return pl.pallas_call(
        _logsoftmax_body,
        grid=(_B,),
        in_specs=[
            pl.BlockSpec((1, _T, _V), lambda b: (b, 0, 0)),
            pl.BlockSpec((1, _U, _V), lambda b: (b, 0, 0)),
        ],
        out_specs=[
            pl.BlockSpec((1, _T, _V), lambda b: (b, 0, 0)),
            pl.BlockSpec((1, _U, _V), lambda b: (b, 0, 0)),
        ],
        out_shape=[
            jax.ShapeDtypeStruct((_B, _T, _V), tn_out.dtype),
            jax.ShapeDtypeStruct((_B, _U, _V), pn_out.dtype),
        ],
    )(tn_out, pn_out)


def _expand_compute(ob, pn_v, tn_v, u):
    pnregs = tuple(pn_v[u, pl.ds(vc * _LANES, _LANES)] for vc in range(_NVC))

    def tbody(t, regs):
        for vc in range(_NVC):
            sl = pl.ds(vc * _LANES, _LANES)
            ob[t, sl] = tn_v[t, sl] + regs[vc]
        return regs

    lax.fori_loop(0, _TQ, tbody, pnregs)


def _expand_body(tn_ls, pn_ls, out, pn_v, tn_v, ob0, ob1, sem0, sem1):
    wid = lax.axis_index("s") * 2 + lax.axis_index("c")
    b = wid // 8
    r = wid % 8
    u0 = (r // 4) * _UH
    t0 = (r % 4) * _TQ

    pltpu.sync_copy(pn_ls.at[b], pn_v)
    pltpu.sync_copy(tn_ls.at[b, pl.ds(t0, _TQ)], tn_v)

    def upair(j, carry):
        u_a = 2 * j
        u_b = 2 * j + 1

        @pl.when(j > 0)
        def _():
            pltpu.make_async_copy(
                ob0, out.at[b, u0 + u_a - 2, pl.ds(t0, _TQ)], sem0).wait()

        _expand_compute(ob0, pn_v, tn_v, u0 + u_a)
        pltpu.make_async_copy(
            ob0, out.at[b, u0 + u_a, pl.ds(t0, _TQ)], sem0).start()

        @pl.when(j > 0)
        def _():
            pltpu.make_async_copy(
                ob1, out.at[b, u0 + u_b - 2, pl.ds(t0, _TQ)], sem1).wait()

        _expand_compute(ob1, pn_v, tn_v, u0 + u_b)
        pltpu.make_async_copy(
            ob1, out.at[b, u0 + u_b, pl.ds(t0, _TQ)], sem1).start()
        return carry

    lax.fori_loop(0, _UH // 2, upair, 0)

    u_t = _UH - 1
    pltpu.make_async_copy(
        ob0, out.at[b, u0 + u_t - 2, pl.ds(t0, _TQ)], sem0).wait()
    _expand_compute(ob0, pn_v, tn_v, u0 + u_t)
    pltpu.make_async_copy(
        ob0, out.at[b, u0 + u_t, pl.ds(t0, _TQ)], sem0).start()
    pltpu.make_async_copy(
        ob0, out.at[b, u0 + u_t, pl.ds(t0, _TQ)], sem0).wait()
    pltpu.make_async_copy(
        ob1, out.at[b, u0 + u_t - 1, pl.ds(t0, _TQ)], sem1).wait()


def _expand(tn_ls, pn_ls):
    mesh = plsc.VectorSubcoreMesh(core_axis_name="c", subcore_axis_name="s")
    run = pl.kernel(
        _expand_body,
        out_type=jax.ShapeDtypeStruct((_B, _U, _T, _V), tn_ls.dtype),
        mesh=mesh,
        scratch_types=[
            pltpu.VMEM((_U, _V), jnp.float32),
            pltpu.VMEM((_TQ, _V), jnp.float32),
            pltpu.VMEM((_TQ, _V), jnp.float32),
            pltpu.VMEM((_TQ, _V), jnp.float32),
            pltpu.SemaphoreType.DMA,
            pltpu.SemaphoreType.DMA,
        ],
    )
    return run(tn_ls, pn_ls)


def kernel(tn_out, pn_out):
    tn_ls, pn_ls = _log_softmaxes(tn_out, pn_out)
    out_butv = _expand(tn_ls, pn_ls)
    return jnp.swapaxes(out_butv, 1, 2)
